# baseline (speedup 1.0000x reference)
import jax
import jax.numpy as jnp
from jax import lax
from jax.experimental import pallas as pl

_B, _N, _D = 16, 256, 128


_BB = 8


def _gcn_body(x_ref, w_ref, b_ref, o_ref):
    ii = lax.broadcasted_iota(jnp.int32, (_N, _N), 0)
    jj = lax.broadcasted_iota(jnp.int32, (_N, _N), 1)
    fi = (ii + 1).astype(jnp.float32)
    fj = (jj + 1).astype(jnp.float32)
    m = jnp.where(jj <= ii, lax.rsqrt(fi * fj), 0.0).astype(jnp.bfloat16)
    w = w_ref[...].astype(jnp.bfloat16)
    b = b_ref[...]
    for k in range(_BB):
        xk = x_ref[k].astype(jnp.bfloat16)
        t = jnp.dot(m, xk, preferred_element_type=jnp.float32)
        out = jnp.dot(t.astype(jnp.bfloat16), w, preferred_element_type=jnp.float32)
        o_ref[pl.ds(k * _N, _N), :] = jnp.maximum(out + b, 0.0)


def kernel(x, W, bias):
    bias2 = bias.reshape(1, _D)
    out = pl.pallas_call(
        _gcn_body,
        grid=(_B // _BB,),
        in_specs=[
            pl.BlockSpec((_BB, _N, _D), lambda b: (b, 0, 0)),
            pl.BlockSpec((_D, _D), lambda b: (0, 0)),
            pl.BlockSpec((1, _D), lambda b: (0, 0)),
        ],
        out_specs=pl.BlockSpec((_BB * _N, _D), lambda b: (b, 0)),
        out_shape=jax.ShapeDtypeStruct((_B * _N, _D), jnp.float32),
    )(x, W, bias2)
    return out


# speedup vs baseline: 1.4586x; 1.4586x over previous
import jax
import jax.numpy as jnp
from jax import lax
from jax.experimental import pallas as pl

_B, _N, _D = 16, 256, 128


_BB = 8


def _gcn_body(x_ref, w_ref, b_ref, o_ref):
    ii = lax.broadcasted_iota(jnp.int32, (_N, _N), 0)
    jj = lax.broadcasted_iota(jnp.int32, (_N, _N), 1)
    fi = (ii + 1).astype(jnp.float32)
    fj = (jj + 1).astype(jnp.float32)
    m = jnp.where(jj <= ii, lax.rsqrt(fi * fj), 0.0)
    w = w_ref[...]
    b = b_ref[...]
    xw = jnp.dot(
        x_ref[...].reshape(_BB * _N, _D), w, preferred_element_type=jnp.float32
    )
    for k in range(_BB):
        out = jnp.dot(
            m, xw[k * _N:(k + 1) * _N, :], preferred_element_type=jnp.float32
        )
        o_ref[pl.ds(k * _N, _N), :] = jnp.maximum(out + b, 0.0)


def kernel(x, W, bias):
    bias2 = bias.reshape(1, _D)
    out = pl.pallas_call(
        _gcn_body,
        grid=(_B // _BB,),
        in_specs=[
            pl.BlockSpec((_BB, _N, _D), lambda b: (b, 0, 0)),
            pl.BlockSpec((_D, _D), lambda b: (0, 0)),
            pl.BlockSpec((1, _D), lambda b: (0, 0)),
        ],
        out_specs=pl.BlockSpec((_BB * _N, _D), lambda b: (b, 0)),
        out_shape=jax.ShapeDtypeStruct((_B * _N, _D), jnp.float32),
    )(x, W, bias2)
    return out
